# Initial kernel scaffold; baseline (speedup 1.0000x reference)
import jax
import jax.numpy as jnp
from jax.experimental import pallas as pl


def kernel(input, table, max_steps):
    raise NotImplementedError("write your pallas kernel here")


# speedup vs baseline: 1.1749x; 1.1749x over previous
import functools

import jax
import jax.numpy as jnp
from jax import lax
from jax.experimental import pallas as pl
from jax.experimental.pallas import tpu as pltpu
from jax.experimental.pallas import tpu_sc as plsc

NC = 2
NS = 16
NW = NC * NS

CHUNK = 1024
GATHER = 128


def _gather_body(E, rows_per_w, idx_hbm, table_hbm, out_hbm,
                 idx_v, rows_v, sem):
    wid = lax.axis_index("s") * NC + lax.axis_index("c")
    base = wid * rows_per_w

    def chunk_body(c, carry):
        off = base + c * CHUNK
        pltpu.sync_copy(idx_hbm.at[pl.ds(off, CHUNK)], idx_v)
        copies = []
        for j in range(CHUNK // GATHER):
            copies.append(pltpu.async_copy(
                table_hbm.at[idx_v.at[pl.ds(j * GATHER, GATHER)]],
                rows_v.at[pl.ds(j * GATHER, GATHER)],
                sem))

        def cnt_body(i, acc):
            v = idx_v[pl.ds(i * 16, 16)]
            ones = jnp.ones((16,), jnp.int32)
            zer = jnp.zeros((16,), jnp.int32)
            return acc + jnp.sum(jnp.where(v == 0, ones, zer))
        nz = lax.fori_loop(0, CHUNK // 16, cnt_body, jnp.int32(0))

        for cp in copies:
            cp.wait()

        @pl.when(nz > 0)
        def _fix():
            zeros = jnp.zeros((16,), jnp.float32)

            def fix_body(i, carry2):
                v = idx_v[pl.ds(i * 16, 16)]
                m = v == 0
                rowids = lax.iota(jnp.int32, 16) + i * 16
                for col in range(E):
                    colids = jnp.full((16,), col, jnp.int32)
                    plsc.store_scatter(rows_v, [rowids, colids], zeros,
                                       mask=m)
                return carry2
            lax.fori_loop(0, CHUNK // 16, fix_body, jnp.int32(0))

        pltpu.sync_copy(rows_v, out_hbm.at[pl.ds(off, CHUNK)])
        return carry

    lax.fori_loop(0, rows_per_w // CHUNK, chunk_body, jnp.int32(0))


def kernel(input, table, max_steps):
    B, L = input.shape
    V, E = table.shape
    N = B * L
    rows_per_w = N // NW

    idx_flat = input.T.reshape(N).astype(jnp.int32)

    mesh = plsc.VectorSubcoreMesh(core_axis_name="c", subcore_axis_name="s")
    body = functools.partial(_gather_body, E, rows_per_w)
    out = pl.kernel(
        body,
        out_type=jax.ShapeDtypeStruct((N, E), jnp.float32),
        mesh=mesh,
        compiler_params=pltpu.CompilerParams(use_tc_tiling_on_sc=False,
                                             needs_layout_passes=False),
        scratch_types=[
            pltpu.VMEM((CHUNK,), jnp.int32),
            pltpu.VMEM((CHUNK, E), jnp.float32),
            pltpu.SemaphoreType.DMA,
        ],
    )(idx_flat, table)
    return out.reshape(L, B, E)


# speedup vs baseline: 1.2046x; 1.0253x over previous
import functools

import jax
import jax.numpy as jnp
from jax import lax
from jax.experimental import pallas as pl
from jax.experimental.pallas import tpu as pltpu
from jax.experimental.pallas import tpu_sc as plsc

NC = 2
NS = 16
NW = NC * NS

CHUNK = 1280
GATHER = 128


def _count_zeros(idx_v):
    ones = jnp.ones((16,), jnp.int32)
    zer = jnp.zeros((16,), jnp.int32)

    def cnt_body(i, acc):
        v = idx_v[pl.ds(i * 16, 16)]
        return acc + jnp.sum(jnp.where(v == 0, ones, zer))

    return lax.fori_loop(0, CHUNK // 16, cnt_body, jnp.int32(0))


def _fix_zero_rows(E, idx_v, rows_v):
    zeros = jnp.zeros((16,), jnp.float32)

    def fix_body(i, carry):
        v = idx_v[pl.ds(i * 16, 16)]
        m = v == 0
        rowids = lax.iota(jnp.int32, 16) + i * 16
        for col in range(E):
            colids = jnp.full((16,), col, jnp.int32)
            plsc.store_scatter(rows_v, [rowids, colids], zeros, mask=m)
        return carry

    lax.fori_loop(0, CHUNK // 16, fix_body, jnp.int32(0))


def _gather_body(E, rows_per_w, idx_hbm, table_hbm, out_hbm,
                 idx0, idx1, rows0, rows1, sg0, sg1, sw0, sw1):
    wid = lax.axis_index("s") * NC + lax.axis_index("c")
    base = wid * rows_per_w
    nch = rows_per_w // CHUNK

    def load_and_fire(c, ib, rb, sg):
        pltpu.sync_copy(idx_hbm.at[pl.ds(base + c * CHUNK, CHUNK)], ib)
        for j in range(CHUNK // GATHER):
            pltpu.async_copy(
                table_hbm.at[ib.at[pl.ds(j * GATHER, GATHER)]],
                rb.at[pl.ds(j * GATHER, GATHER)], sg)

    def wait_gathers(rb, sg):
        pltpu.make_async_copy(table_hbm.at[pl.ds(0, CHUNK)], rb, sg).wait()

    def wait_write(rb, sw):
        pltpu.make_async_copy(rb, out_hbm.at[pl.ds(base, CHUNK)], sw).wait()

    def process(c, nz, ib, rb, sg, sw):
        wait_gathers(rb, sg)

        @pl.when(nz > 0)
        def _():
            _fix_zero_rows(E, ib, rb)

        pltpu.async_copy(rb, out_hbm.at[pl.ds(base + c * CHUNK, CHUNK)], sw)

    def prefetch(c, first, last, ib, rb, sg, sw):
        @pl.when(jnp.logical_not(jnp.logical_or(first, last)))
        def _():
            wait_write(rb, sw)

        @pl.when(jnp.logical_not(last))
        def _():
            load_and_fire(c, ib, rb, sg)
        return _count_zeros(ib)

    nz0 = prefetch(0, jnp.bool_(True), jnp.bool_(False), idx0, rows0,
                   sg0, sw0)

    def loop_body(i, carry):
        nz0, nz1 = carry
        c0 = 2 * i
        nz1 = prefetch(c0 + 1, i == 0, jnp.bool_(False), idx1, rows1,
                       sg1, sw1)
        process(c0, nz0, idx0, rows0, sg0, sw0)
        nz0 = prefetch(c0 + 2, jnp.bool_(False), i == nch // 2 - 1,
                       idx0, rows0, sg0, sw0)
        process(c0 + 1, nz1, idx1, rows1, sg1, sw1)
        return nz0, nz1

    lax.fori_loop(0, nch // 2, loop_body, (nz0, nz0))

    wait_write(rows0, sw0)
    wait_write(rows1, sw1)


def kernel(input, table, max_steps):
    B, L = input.shape
    V, E = table.shape
    N = B * L
    rows_per_w = N // NW

    idx_flat = input.T.reshape(N).astype(jnp.int32)

    mesh = plsc.VectorSubcoreMesh(core_axis_name="c", subcore_axis_name="s")
    body = functools.partial(_gather_body, E, rows_per_w)
    out = pl.kernel(
        body,
        out_type=jax.ShapeDtypeStruct((N, E), jnp.float32),
        mesh=mesh,
        compiler_params=pltpu.CompilerParams(use_tc_tiling_on_sc=False,
                                             needs_layout_passes=False),
        scratch_types=[
            pltpu.VMEM((CHUNK,), jnp.int32),
            pltpu.VMEM((CHUNK,), jnp.int32),
            pltpu.VMEM((CHUNK, E), jnp.float32),
            pltpu.VMEM((CHUNK, E), jnp.float32),
            pltpu.SemaphoreType.DMA,
            pltpu.SemaphoreType.DMA,
            pltpu.SemaphoreType.DMA,
            pltpu.SemaphoreType.DMA,
        ],
    )(idx_flat, table)
    return out.reshape(L, B, E)
